# baseline (speedup 1.0000x reference)
import jax
import jax.numpy as jnp
from jax.experimental import pallas as pl
from jax.experimental.pallas import tpu as pltpu

_B = 4096
_C = 20000
_D = 128
_CORES = 2
_ROWS = _B // _CORES
_UNROLL = 128
_NACC = 8


def _center_loss_kernel(labels_ref, x_ref, centers_ref, out_ref):
    base = pl.program_id(0) * _ROWS

    def body(o, accs):
        accs = list(accs)
        r = o * _UNROLL
        for j in range(0, _UNROLL, _NACC):
            ds = []
            for k in range(_NACC):
                i = r + j + k
                ds.append(x_ref[i, 0] - centers_ref[labels_ref[base + i], 0])
            for k in range(_NACC):
                accs[k] = accs[k] + ds[k] * ds[k]
        return tuple(accs)

    z = jnp.zeros((_D,), jnp.float32)
    accs = jax.lax.fori_loop(0, _ROWS // _UNROLL, body, (z,) * _NACC)
    total = accs[0]
    for k in range(1, _NACC):
        total = total + accs[k]
    out_ref[0, 0, :] = total


@jax.jit
def kernel(x, labels, centers):
    labels32 = labels.astype(jnp.int32)
    x3 = x.reshape(_B, 1, _D)
    c3 = centers.reshape(_C, 1, _D)
    grid_spec = pltpu.PrefetchScalarGridSpec(
        num_scalar_prefetch=1,
        grid=(_CORES,),
        in_specs=[
            pl.BlockSpec((_ROWS, 1, _D), lambda i, lbl: (i, 0, 0)),
            pl.BlockSpec((_C, 1, _D), lambda i, lbl: (0, 0, 0)),
        ],
        out_specs=pl.BlockSpec((1, 1, _D), lambda i, lbl: (i, 0, 0)),
    )
    partials = pl.pallas_call(
        _center_loss_kernel,
        grid_spec=grid_spec,
        out_shape=jax.ShapeDtypeStruct((_CORES, 1, _D), jnp.float32),
        compiler_params=pltpu.CompilerParams(
            dimension_semantics=("parallel",),
        ),
    )(labels32, x3, c3)
    return jnp.sum(partials) / jnp.float32(_B * _C)


# speedup vs baseline: 1.0413x; 1.0413x over previous
import jax
import jax.numpy as jnp
from jax.experimental import pallas as pl
from jax.experimental.pallas import tpu as pltpu

_B = 4096
_C = 20000
_D = 128
_CORES = 2
_ROWS = _B // _CORES
_CHUNK = 128


def _center_loss_kernel(labels_ref, x_ref, centers_ref, out_ref, tile_ref):
    base = pl.program_id(0) * _ROWS

    def body(o, acc):
        r = o * _CHUNK
        for j in range(_CHUNK):
            tile_ref[j] = centers_ref[labels_ref[base + r + j]]
        xs = x_ref[pl.ds(r, _CHUNK), :, :]
        d = xs - tile_ref[...]
        return acc + jnp.sum(d * d, axis=0)

    acc = jax.lax.fori_loop(
        0, _ROWS // _CHUNK, body, jnp.zeros((1, _D), jnp.float32)
    )
    out_ref[0, 0, :] = acc[0]


@jax.jit
def kernel(x, labels, centers):
    labels32 = labels.astype(jnp.int32)
    x3 = x.reshape(_B, 1, _D)
    c3 = centers.reshape(_C, 1, _D)
    grid_spec = pltpu.PrefetchScalarGridSpec(
        num_scalar_prefetch=1,
        grid=(_CORES,),
        in_specs=[
            pl.BlockSpec((_ROWS, 1, _D), lambda i, lbl: (i, 0, 0)),
            pl.BlockSpec((_C, 1, _D), lambda i, lbl: (0, 0, 0)),
        ],
        out_specs=pl.BlockSpec((1, 1, _D), lambda i, lbl: (i, 0, 0)),
        scratch_shapes=[pltpu.VMEM((_CHUNK, 1, _D), jnp.float32)],
    )
    partials = pl.pallas_call(
        _center_loss_kernel,
        grid_spec=grid_spec,
        out_shape=jax.ShapeDtypeStruct((_CORES, 1, _D), jnp.float32),
        compiler_params=pltpu.CompilerParams(
            dimension_semantics=("parallel",),
        ),
    )(labels32, x3, c3)
    return jnp.sum(partials) / jnp.float32(_B * _C)
